# baseline (speedup 1.0000x reference)
import functools

import jax
import jax.numpy as jnp
from jax import lax
from jax.experimental import pallas as pl
from jax.experimental.pallas import tpu as pltpu
from jax.experimental.pallas import tpu_sc as plsc

Q = 1024
K = 8192
D = 2048
TOPK = 16

_BQ = 1024
_BK = 1024


def _sims_body(q_ref, m_ref, out_ref):
    q = q_ref[...]
    m = m_ref[...]
    dots = lax.dot_general(q, m, (((1,), (1,)), ((), ())),
                           preferred_element_type=jnp.float32)
    qn = jnp.sqrt(jnp.sum(q * q, axis=1))
    mn = jnp.sqrt(jnp.sum(m * m, axis=1))
    denom = jnp.maximum(qn[:, None] * mn[None, :], jnp.float32(1e-8))
    out_ref[...] = dots / denom


def _sims(query, memory_slots):
    return pl.pallas_call(
        _sims_body,
        grid=(Q // _BQ, K // _BK),
        in_specs=[
            pl.BlockSpec((_BQ, D), lambda i, j: (i, 0)),
            pl.BlockSpec((_BK, D), lambda i, j: (j, 0)),
        ],
        out_specs=pl.BlockSpec((_BQ, _BK), lambda i, j: (i, j)),
        out_shape=jax.ShapeDtypeStruct((Q, K), jnp.float32),
    )(query, memory_slots)


_TQ = 256
_NP = 16
_GW = K // _NP


def _iter_topk(vals, gids):
    return _iter_topk_multi([vals], [gids])


def _iter_topk_multi(vs, gs):
    neg_inf = jnp.float32(-jnp.inf)
    big = jnp.int32(2**31 - 1)
    vs = list(vs)
    cols = []
    w16 = None
    for _ in range(TOPK):
        m = jnp.max(vs[0], axis=1, keepdims=True)
        for v in vs[1:]:
            m = jnp.maximum(m, jnp.max(v, axis=1, keepdims=True))
        g = jnp.full_like(m, big, dtype=jnp.int32)
        for v, gd in zip(vs, gs):
            g = jnp.minimum(
                g, jnp.min(jnp.where(v == m, gd, big), axis=1, keepdims=True))
        cols.append(g)
        w16 = m
        vs = [jnp.where(gd == g, neg_inf, v) for v, gd in zip(vs, gs)]
    return jnp.concatenate(cols, axis=1), w16


def _topk_body(s_ref, idx_ref):
    neg_inf = jnp.float32(-jnp.inf)
    lane = lax.broadcasted_iota(jnp.int32, (_TQ, _GW), 1)
    v1 = s_ref[:, 0:_GW]
    g1 = lane
    v2 = jnp.full((_TQ, _GW), neg_inf)
    v3 = v2
    g2 = jnp.zeros((_TQ, _GW), jnp.int32)
    g3 = g2
    for p in range(1, _NP):
        x = s_ref[:, p * _GW:(p + 1) * _GW]
        gx = lane + jnp.int32(p * _GW)
        b1 = x > v1
        b2 = x > v2
        b3 = x > v3
        v3 = jnp.where(b2, v2, jnp.where(b3, x, v3))
        g3 = jnp.where(b2, g2, jnp.where(b3, gx, g3))
        v2 = jnp.where(b1, v1, jnp.where(b2, x, v2))
        g2 = jnp.where(b1, g1, jnp.where(b2, gx, g2))
        v1 = jnp.where(b1, x, v1)
        g1 = jnp.where(b1, gx, g1)
    idx, w16 = _iter_topk_multi([v1, v2, v3], [g1, g2, g3])
    idx_ref[...] = idx
    fb = jnp.any(jnp.max(v3, axis=1, keepdims=True) >= w16)

    @pl.when(fb)
    def _fallback():
        gidx = lax.broadcasted_iota(jnp.int32, (_TQ, K), 1)
        idx_full, _ = _iter_topk(s_ref[...], gidx)
        idx_ref[...] = idx_full


def _topk(sims):
    return pl.pallas_call(
        _topk_body,
        grid=(Q // _TQ,),
        in_specs=[pl.BlockSpec((_TQ, K), lambda i: (i, 0))],
        out_specs=pl.BlockSpec((_TQ, TOPK), lambda i: (i, 0)),
        out_shape=jax.ShapeDtypeStruct((Q, TOPK), jnp.int32),
    )(sims)


_CH = 16


def _sc_gather(table, idx_flat):
    B = idx_flat.shape[0]
    NC, NS = 2, 16
    NW = NC * NS
    b_per_w = B // NW
    n_ch = b_per_w // _CH
    mesh = plsc.VectorSubcoreMesh(core_axis_name="c", subcore_axis_name="s")

    @functools.partial(
        pl.kernel,
        mesh=mesh,
        out_type=jax.ShapeDtypeStruct((B, D), jnp.float32),
        scratch_types=[
            pltpu.VMEM((b_per_w,), jnp.int32),
            pltpu.VMEM((_CH, D), jnp.float32),
            pltpu.VMEM((_CH, D), jnp.float32),
            pltpu.VMEM((_CH, D), jnp.float32),
            pltpu.SemaphoreType.DMA,
            pltpu.SemaphoreType.DMA,
            pltpu.SemaphoreType.DMA,
            pltpu.SemaphoreType.DMA,
            pltpu.SemaphoreType.DMA,
            pltpu.SemaphoreType.DMA,
        ],
    )
    def k(table_hbm, idx_hbm, out_hbm, idx_v,
          buf0, buf1, buf2, g0, g1, g2, s0, s1, s2):
        wid = lax.axis_index("s") * NC + lax.axis_index("c")
        base = wid * b_per_w
        pltpu.sync_copy(idx_hbm.at[pl.ds(base, b_per_w)], idx_v)
        bufs = (buf0, buf1, buf2)
        gsems = (g0, g1, g2)
        ssems = (s0, s1, s2)

        def gather(c):
            return pltpu.async_copy(
                table_hbm.at[idx_v.at[pl.ds(c * _CH, _CH)]],
                bufs[c % 3], gsems[c % 3])

        def scatter(c):
            return pltpu.async_copy(
                bufs[c % 3], out_hbm.at[pl.ds(base + c * _CH, _CH)],
                ssems[c % 3])

        pend_g = {0: gather(0), 1: gather(1)}
        pend_s = {}
        for c in range(n_ch):
            pend_g.pop(c).wait()
            pend_s[c] = scatter(c)
            if c + 2 < n_ch:
                if c >= 1:
                    pend_s.pop(c - 1).wait()
                pend_g[c + 2] = gather(c + 2)
        pend_s.pop(n_ch - 2).wait()
        pend_s.pop(n_ch - 1).wait()

    return k(table, idx_flat)


def kernel(query, memory_slots, top_k):
    sims = _sims(query, memory_slots)
    idx = _topk(sims)
    rows = _sc_gather(memory_slots, idx.reshape(-1))
    return rows.reshape(Q, TOPK, D)


# speedup vs baseline: 1.3597x; 1.3597x over previous
import functools

import jax
import jax.numpy as jnp
from jax import lax
from jax.experimental import pallas as pl
from jax.experimental.pallas import tpu as pltpu
from jax.experimental.pallas import tpu_sc as plsc

Q = 1024
K = 8192
D = 2048
TOPK = 16

_BQ = 1024
_BK = 1024


def _sims_body(q_ref, m_ref, out_ref):
    q = q_ref[...]
    m = m_ref[...]
    dots = lax.dot_general(q, m, (((1,), (1,)), ((), ())),
                           preferred_element_type=jnp.float32)
    qn = jnp.sqrt(jnp.sum(q * q, axis=1))
    mn = jnp.sqrt(jnp.sum(m * m, axis=1))
    denom = jnp.maximum(qn[:, None] * mn[None, :], jnp.float32(1e-8))
    out_ref[...] = dots / denom


def _sims(query, memory_slots):
    return pl.pallas_call(
        _sims_body,
        grid=(Q // _BQ, K // _BK),
        in_specs=[
            pl.BlockSpec((_BQ, D), lambda i, j: (i, 0)),
            pl.BlockSpec((_BK, D), lambda i, j: (j, 0)),
        ],
        out_specs=pl.BlockSpec((_BQ, _BK), lambda i, j: (i, j)),
        out_shape=jax.ShapeDtypeStruct((Q, K), jnp.float32),
    )(query, memory_slots)


_TQ = 128
_NP = 16
_GW = K // _NP


def _iter_topk(vals, gids):
    return _iter_topk_multi([vals], [gids])


def _iter_topk_multi(vs, gs):
    neg_inf = jnp.float32(-jnp.inf)
    big = jnp.int32(2**31 - 1)
    vs = list(vs)
    cols = []
    w16 = None
    for _ in range(TOPK):
        m = jnp.max(vs[0], axis=1, keepdims=True)
        for v in vs[1:]:
            m = jnp.maximum(m, jnp.max(v, axis=1, keepdims=True))
        g = jnp.full_like(m, big, dtype=jnp.int32)
        for v, gd in zip(vs, gs):
            g = jnp.minimum(
                g, jnp.min(jnp.where(v == m, gd, big), axis=1, keepdims=True))
        cols.append(g)
        w16 = m
        vs = [jnp.where(gd == g, neg_inf, v) for v, gd in zip(vs, gs)]
    return jnp.concatenate(cols, axis=1), w16


def _topk_body(s_ref, idx_ref):
    neg_inf = jnp.float32(-jnp.inf)
    lane = lax.broadcasted_iota(jnp.int32, (_TQ, _GW), 1)
    v1 = s_ref[:, 0:_GW]
    g1 = lane
    v2 = jnp.full((_TQ, _GW), neg_inf)
    v3 = v2
    g2 = jnp.zeros((_TQ, _GW), jnp.int32)
    g3 = g2
    for p in range(1, _NP):
        x = s_ref[:, p * _GW:(p + 1) * _GW]
        gx = lane + jnp.int32(p * _GW)
        b1 = x > v1
        b2 = x > v2
        b3 = x > v3
        v3 = jnp.where(b2, v2, jnp.where(b3, x, v3))
        g3 = jnp.where(b2, g2, jnp.where(b3, gx, g3))
        v2 = jnp.where(b1, v1, jnp.where(b2, x, v2))
        g2 = jnp.where(b1, g1, jnp.where(b2, gx, g2))
        v1 = jnp.where(b1, x, v1)
        g1 = jnp.where(b1, gx, g1)
    idx, w16 = _iter_topk_multi([v1, v2, v3], [g1, g2, g3])
    idx_ref[...] = idx
    fb = jnp.any(jnp.max(v3, axis=1, keepdims=True) >= w16)

    @pl.when(fb)
    def _fallback():
        gidx = lax.broadcasted_iota(jnp.int32, (_TQ, K), 1)
        idx_full, _ = _iter_topk(s_ref[...], gidx)
        idx_ref[...] = idx_full


def _topk(sims):
    return pl.pallas_call(
        _topk_body,
        grid=(Q // _TQ,),
        in_specs=[pl.BlockSpec((_TQ, K), lambda i: (i, 0))],
        out_specs=pl.BlockSpec((_TQ, TOPK), lambda i: (i, 0)),
        out_shape=jax.ShapeDtypeStruct((Q, TOPK), jnp.int32),
    )(sims)


_CH = 16


def _sc_gather(table, idx_flat):
    B = idx_flat.shape[0]
    NC, NS = 2, 16
    NW = NC * NS
    b_per_w = B // NW
    n_ch = b_per_w // _CH
    mesh = plsc.VectorSubcoreMesh(core_axis_name="c", subcore_axis_name="s")

    @functools.partial(
        pl.kernel,
        mesh=mesh,
        out_type=jax.ShapeDtypeStruct((B, D), jnp.float32),
        scratch_types=[
            pltpu.VMEM((b_per_w,), jnp.int32),
            pltpu.VMEM((_CH, D), jnp.float32),
            pltpu.VMEM((_CH, D), jnp.float32),
            pltpu.VMEM((_CH, D), jnp.float32),
            pltpu.SemaphoreType.DMA,
            pltpu.SemaphoreType.DMA,
            pltpu.SemaphoreType.DMA,
            pltpu.SemaphoreType.DMA,
            pltpu.SemaphoreType.DMA,
            pltpu.SemaphoreType.DMA,
        ],
    )
    def k(table_hbm, idx_hbm, out_hbm, idx_v,
          buf0, buf1, buf2, g0, g1, g2, s0, s1, s2):
        wid = lax.axis_index("s") * NC + lax.axis_index("c")
        base = wid * b_per_w
        pltpu.sync_copy(idx_hbm.at[pl.ds(base, b_per_w)], idx_v)
        bufs = (buf0, buf1, buf2)
        gsems = (g0, g1, g2)
        ssems = (s0, s1, s2)

        def gather(c):
            return pltpu.async_copy(
                table_hbm.at[idx_v.at[pl.ds(c * _CH, _CH)]],
                bufs[c % 3], gsems[c % 3])

        def scatter(c):
            return pltpu.async_copy(
                bufs[c % 3], out_hbm.at[pl.ds(base + c * _CH, _CH)],
                ssems[c % 3])

        pend_g = {0: gather(0), 1: gather(1)}
        pend_s = {}
        for c in range(n_ch):
            pend_g.pop(c).wait()
            pend_s[c] = scatter(c)
            if c + 2 < n_ch:
                if c >= 1:
                    pend_s.pop(c - 1).wait()
                pend_g[c + 2] = gather(c + 2)
        pend_s.pop(n_ch - 2).wait()
        pend_s.pop(n_ch - 1).wait()

    return k(table, idx_flat)


def kernel(query, memory_slots, top_k):
    sims = _sims(query, memory_slots)
    idx = _topk(sims)
    rows = _sc_gather(memory_slots, idx.reshape(-1))
    return rows.reshape(Q, TOPK, D)


# speedup vs baseline: 1.3642x; 1.0033x over previous
import functools

import jax
import jax.numpy as jnp
from jax import lax
from jax.experimental import pallas as pl
from jax.experimental.pallas import tpu as pltpu
from jax.experimental.pallas import tpu_sc as plsc

Q = 1024
K = 8192
D = 2048
TOPK = 16

_BQ = 1024
_BK = 1024


def _sims_body(q_ref, m_ref, out_ref):
    q = q_ref[...]
    m = m_ref[...]
    dots = lax.dot_general(q, m, (((1,), (1,)), ((), ())),
                           preferred_element_type=jnp.float32)
    qn = jnp.sqrt(jnp.sum(q * q, axis=1))
    mn = jnp.sqrt(jnp.sum(m * m, axis=1))
    denom = jnp.maximum(qn[:, None] * mn[None, :], jnp.float32(1e-8))
    out_ref[...] = dots / denom


def _sims(query, memory_slots):
    return pl.pallas_call(
        _sims_body,
        grid=(Q // _BQ, K // _BK),
        in_specs=[
            pl.BlockSpec((_BQ, D), lambda i, j: (i, 0)),
            pl.BlockSpec((_BK, D), lambda i, j: (j, 0)),
        ],
        out_specs=pl.BlockSpec((_BQ, _BK), lambda i, j: (i, j)),
        out_shape=jax.ShapeDtypeStruct((Q, K), jnp.float32),
    )(query, memory_slots)


_TQ = 128
_NP = 16
_GW = K // _NP


def _iter_topk(vals, gids):
    return _iter_topk_multi([vals], [gids])


def _iter_topk_multi(vs, gs):
    neg_inf = jnp.float32(-jnp.inf)
    big = jnp.int32(2**31 - 1)
    vs = list(vs)
    cols = []
    w16 = None
    for _ in range(TOPK):
        m = jnp.max(vs[0], axis=1, keepdims=True)
        for v in vs[1:]:
            m = jnp.maximum(m, jnp.max(v, axis=1, keepdims=True))
        g = jnp.full_like(m, big, dtype=jnp.int32)
        for v, gd in zip(vs, gs):
            g = jnp.minimum(
                g, jnp.min(jnp.where(v == m, gd, big), axis=1, keepdims=True))
        cols.append(g)
        w16 = m
        vs = [jnp.where(gd == g, neg_inf, v) for v, gd in zip(vs, gs)]
    return jnp.concatenate(cols, axis=1), w16


def _topk_body(s_ref, idx_ref):
    neg_inf = jnp.float32(-jnp.inf)
    lane = lax.broadcasted_iota(jnp.int32, (_TQ, _GW), 1)
    v1 = s_ref[:, 0:_GW]
    g1 = lane
    v2 = jnp.full((_TQ, _GW), neg_inf)
    v3 = v2
    g2 = jnp.zeros((_TQ, _GW), jnp.int32)
    g3 = g2
    for p in range(1, _NP):
        x = s_ref[:, p * _GW:(p + 1) * _GW]
        gx = lane + jnp.int32(p * _GW)
        b1 = x > v1
        b2 = x > v2
        b3 = x > v3
        v3 = jnp.where(b2, v2, jnp.where(b3, x, v3))
        g3 = jnp.where(b2, g2, jnp.where(b3, gx, g3))
        v2 = jnp.where(b1, v1, jnp.where(b2, x, v2))
        g2 = jnp.where(b1, g1, jnp.where(b2, gx, g2))
        v1 = jnp.where(b1, x, v1)
        g1 = jnp.where(b1, gx, g1)
    idx, w16 = _iter_topk_multi([v1, v2, v3], [g1, g2, g3])
    idx_ref[...] = idx
    fb = jnp.any(jnp.max(v3, axis=1, keepdims=True) >= w16)

    @pl.when(fb)
    def _fallback():
        gidx = lax.broadcasted_iota(jnp.int32, (_TQ, K), 1)
        idx_full, _ = _iter_topk(s_ref[...], gidx)
        idx_ref[...] = idx_full


def _topk(sims):
    return pl.pallas_call(
        _topk_body,
        grid=(Q // _TQ,),
        in_specs=[pl.BlockSpec((_TQ, K), lambda i: (i, 0))],
        out_specs=pl.BlockSpec((_TQ, TOPK), lambda i: (i, 0)),
        out_shape=jax.ShapeDtypeStruct((Q, TOPK), jnp.int32),
    )(sims)


_CH = 16


def _sc_gather(table, idx_flat):
    B = idx_flat.shape[0]
    NC, NS = 2, 16
    NW = NC * NS
    b_per_w = B // NW
    n_ch = b_per_w // _CH
    mesh = plsc.VectorSubcoreMesh(core_axis_name="c", subcore_axis_name="s")

    @functools.partial(
        pl.kernel,
        mesh=mesh,
        out_type=jax.ShapeDtypeStruct((B, D), jnp.float32),
        scratch_types=[
            pltpu.VMEM((b_per_w,), jnp.int32),
            pltpu.VMEM((_CH, D), jnp.float32),
            pltpu.VMEM((_CH, D), jnp.float32),
            pltpu.VMEM((_CH, D), jnp.float32),
            pltpu.SemaphoreType.DMA,
            pltpu.SemaphoreType.DMA,
            pltpu.SemaphoreType.DMA,
            pltpu.SemaphoreType.DMA,
            pltpu.SemaphoreType.DMA,
            pltpu.SemaphoreType.DMA,
        ],
    )
    def k(table_hbm, idx_hbm, out_hbm, idx_v,
          buf0, buf1, buf2, g0, g1, g2, s0, s1, s2):
        wid = lax.axis_index("s") * NC + lax.axis_index("c")
        base = wid * b_per_w
        pltpu.sync_copy(idx_hbm.at[pl.ds(base, b_per_w)], idx_v)
        bufs = (buf0, buf1, buf2)
        gsems = (g0, g1, g2)
        ssems = (s0, s1, s2)

        def gather(c):
            return pltpu.async_copy(
                table_hbm.at[idx_v.at[pl.ds(c * _CH, _CH)]],
                bufs[c % 3], gsems[c % 3])

        def scatter(c):
            return pltpu.async_copy(
                bufs[c % 3], out_hbm.at[pl.ds(base + c * _CH, _CH)],
                ssems[c % 3])

        pend_g = {0: gather(0), 1: gather(1)}
        pend_s = {}
        for c in range(n_ch):
            pend_g.pop(c).wait()
            pend_s[c] = scatter(c)
            if c + 2 < n_ch:
                if c >= 1:
                    pend_s.pop(c - 1).wait()
                pend_g[c + 2] = gather(c + 2)
        for c in sorted(pend_s):
            pend_s[c].wait()

    return k(table, idx_flat)


def kernel(query, memory_slots, top_k):
    sims = _sims(query, memory_slots)
    idx = _topk(sims)
    rows = _sc_gather(memory_slots, idx.reshape(-1))
    return rows.reshape(Q, TOPK, D)
